# baseline (speedup 1.0000x reference)
import functools
import jax
import jax.numpy as jnp
from jax import lax
from jax.experimental import pallas as pl
from jax.experimental.pallas import tpu as pltpu

NROWS = 64
NCOLS = 1_000_000
BLK = 16_384
GRID = (NCOLS + BLK - 1) // BLK
GROUPS = BLK // 128

_NOISE = None


def _gumbel_noise():
    global _NOISE
    if _NOISE is None:
        def make():
            key = jax.random.key(42)
            u = jax.random.uniform(key, (NROWS, NCOLS), dtype=jnp.float32,
                                   minval=1e-7, maxval=1.0 - 1e-7)
            return -jnp.log(-jnp.log(u))
        _NOISE = jax.jit(make)()
    return _NOISE


def _tc_body(lref, gref, outref, rmax, ridx):
    i = pl.program_id(0)

    @pl.when(i == 0)
    def _():
        rmax[...] = jnp.full((NROWS, 128), -jnp.inf, jnp.float32)
        ridx[...] = jnp.zeros((NROWS, 128), jnp.int32)

    base = i * BLK
    lane = jax.lax.broadcasted_iota(jnp.int32, (NROWS, 128), 1)

    rm = rmax[...]
    ri = ridx[...]

    def group(g, car):
        rm, ri = car
        off = g * 128
        v = lref[:, pl.ds(off, 128)] + gref[:, pl.ds(off, 128)]
        col = (base + off) + lane
        valid = col < NCOLS
        v = jnp.where(valid, v, -jnp.inf)
        m = v > rm
        rm = jnp.where(m, v, rm)
        ri = jnp.where(m, col, ri)
        return rm, ri

    rm, ri = lax.fori_loop(0, GROUPS, group, (rm, ri))
    rmax[...] = rm
    ridx[...] = ri

    @pl.when(i == GRID - 1)
    def _():
        mval = jnp.max(rm, axis=1, keepdims=True)
        cand = jnp.where(rm == mval, ri, jnp.int32(2**31 - 1))
        outref[...] = jnp.min(cand, axis=1, keepdims=True)


_tc_argmax = pl.pallas_call(
    _tc_body,
    grid=(GRID,),
    in_specs=[
        pl.BlockSpec((NROWS, BLK), lambda i: (0, i)),
        pl.BlockSpec((NROWS, BLK), lambda i: (0, i)),
    ],
    out_specs=pl.BlockSpec((NROWS, 1), lambda i: (0, 0)),
    out_shape=jax.ShapeDtypeStruct((NROWS, 1), jnp.int32),
    scratch_shapes=[
        pltpu.VMEM((NROWS, 128), jnp.float32),
        pltpu.VMEM((NROWS, 128), jnp.int32),
    ],
)


def kernel(logits):
    out = _tc_argmax(logits, _gumbel_noise())
    return out.reshape(NROWS)
